# Initial kernel scaffold; baseline (speedup 1.0000x reference)
import jax
import jax.numpy as jnp
from jax.experimental import pallas as pl


def kernel(x, params):
    raise NotImplementedError("write your pallas kernel here")


# speedup vs baseline: 1.0046x; 1.0046x over previous
import jax
import jax.numpy as jnp
from jax.experimental import pallas as pl

_EMBED_DIM = 32
_NUM_EMB = 8192
_BETA = 0.25


def _conv(x, w, b, stride, pad):
    y = jax.lax.conv_general_dilated(
        x, w, (stride, stride), ((pad, pad), (pad, pad)),
        dimension_numbers=('NCHW', 'OIHW', 'NCHW'))
    return y + b[None, :, None, None]


def _convT(x, w, b):
    y = jax.lax.conv_transpose(x, w, (2, 2), 'VALID',
                               dimension_numbers=('NCHW', 'OIHW', 'NCHW'))
    return y + b[None, :, None, None]


def _block(x, p):
    x = jax.nn.relu(_conv(x, p['w1'], p['b1'], 1, 1))
    x = jax.nn.relu(_conv(x, p['w2'], p['b2'], 1, 1))
    return x


def _maxpool(x):
    return jax.lax.reduce_window(x, -jnp.inf, jax.lax.max,
                                 (1, 1, 2, 2), (1, 1, 2, 2), 'VALID')


def _vq_kernel(n_tok, z_ref, cb_ref, zq_ref, loss_ref):
    z = z_ref[...]
    cb = cb_ref[...]
    pad = z.shape[0]
    zn = jnp.sum(z * z, axis=1, keepdims=True)
    cn = jnp.sum(cb * cb, axis=1)[None, :]
    dot = jax.lax.dot_general(z, cb, (((1,), (1,)), ((), ())),
                              preferred_element_type=jnp.float32)
    d2 = (zn + cn) - 2.0 * dot
    d2 = jnp.maximum(d2, 0.0)
    mn = jnp.min(d2, axis=1, keepdims=True)
    lane = jax.lax.broadcasted_iota(jnp.int32, d2.shape, 1)
    idx = jnp.min(jnp.where(d2 == mn, lane, _NUM_EMB), axis=1)
    oh = (lane == idx[:, None]).astype(jnp.float32)
    zq = jax.lax.dot_general(oh, cb, (((1,), (0,)), ((), ())),
                             preferred_element_type=jnp.float32)
    zq_ref[...] = zq
    mask = (jax.lax.broadcasted_iota(jnp.int32, (pad, 1), 0) < n_tok
            ).astype(jnp.float32)
    diff = (zq - z) * mask
    m = jnp.sum(diff * diff, axis=0, keepdims=True) / (n_tok * _EMBED_DIM)
    m = jnp.sum(m, axis=1, keepdims=True)
    loss_ref[...] = m + _BETA * m


def _vq(z_flat, cb):
    n_tok = z_flat.shape[0]
    padded = max(8, -(-n_tok // 8) * 8)
    zp = jnp.pad(z_flat, ((0, padded - n_tok), (0, 0)))
    import functools
    zq_pad, loss = pl.pallas_call(
        functools.partial(_vq_kernel, n_tok),
        out_shape=(jax.ShapeDtypeStruct((padded, _EMBED_DIM), jnp.float32),
                   jax.ShapeDtypeStruct((1, 1), jnp.float32)),
    )(zp, cb)
    return zq_pad[:n_tok], loss[0, 0]


def kernel(x, params):
    h = x
    for p in params['enc']:
        h = _maxpool(_block(h, p))
    z_e = _conv(h, params['pre_w'], params['pre_b'], 1, 0)
    B, C, H, W = z_e.shape
    z_e_flat = jnp.transpose(z_e, (0, 2, 3, 1)).reshape(B * H * W, C)
    z_q, q_loss = _vq(z_e_flat, params['codebook'])
    latent = jnp.transpose(z_q.reshape(B, H, W, C), (0, 3, 1, 2))
    z = _conv(latent, params['post_w'], params['post_b'], 1, 0)
    for p in params['dec']:
        z = _block(_convT(z, p['wt'], p['bt']), p)
    x_reconst = jnp.tanh(_convT(z, params['dec_final_w'],
                                params['dec_final_b']))
    return (x_reconst, latent, q_loss)
